# baseline (speedup 1.0000x reference)
import functools

import jax
import jax.numpy as jnp
from jax import lax
from jax.experimental import pallas as pl
from jax.experimental.pallas import tpu as pltpu
from jax.experimental.pallas import tpu_sc as plsc

KD = 128
EMB = 64
PLEN = 200
POOL = 1000
TOPK = 2
B = 1024
POOL_PAD = 1024

NC, NS = 2, 16
NW = NC * NS
EHALF = 16
NUNITS = PLEN * (EMB // EHALF)
QG = B // 16


def _topk_body(x_ref, k_ref, idx_ref):
    x = x_ref[...]
    k = k_ref[...]
    ksq = jnp.sum(k * k, axis=1, keepdims=True)
    kn = k * lax.rsqrt(jnp.maximum(ksq, 1e-12))
    s = lax.dot_general(x, kn, (((1,), (1,)), ((), ())),
                        precision=lax.Precision.HIGHEST,
                        preferred_element_type=jnp.float32)
    cols = lax.broadcasted_iota(jnp.int32, (B, POOL_PAD), 1)
    s = jnp.where(cols >= POOL, jnp.inf, s)
    m1 = jnp.min(s, axis=1, keepdims=True)
    i1 = jnp.min(jnp.where(s == m1, cols, POOL_PAD), axis=1, keepdims=True)
    s2 = jnp.where(cols == i1, jnp.inf, s)
    m2 = jnp.min(s2, axis=1, keepdims=True)
    i2 = jnp.min(jnp.where(s2 == m2, cols, POOL_PAD), axis=1, keepdims=True)
    idx_ref[...] = jnp.concatenate([i1, i2], axis=1)


_topk = pl.pallas_call(
    _topk_body,
    out_shape=jax.ShapeDtypeStruct((B, TOPK), jnp.int32),
)


@functools.cache
def _make_sc_gather():
    mesh = plsc.VectorSubcoreMesh(core_axis_name="c", subcore_axis_name="s")

    @functools.partial(
        pl.kernel,
        mesh=mesh,
        compiler_params=pltpu.CompilerParams(needs_layout_passes=False),
        out_type=jax.ShapeDtypeStruct((TOPK * PLEN, EMB, B), jnp.float32),
        scratch_types=[
            pltpu.VMEM((TOPK * B,), jnp.int32),
            pltpu.VMEM((TOPK * B,), jnp.int32),
            pltpu.VMEM((2 * EHALF, POOL), jnp.float32),
            pltpu.VMEM((2 * TOPK * EHALF, B), jnp.float32),
            pltpu.SemaphoreType.DMA,
            pltpu.SemaphoreType.DMA,
            pltpu.SemaphoreType.DMA,
            pltpu.SemaphoreType.DMA,
        ],
    )
    def _sc_gather(idx_hbm, pvt_hbm, out_hbm, idx_v, jv_v, in_v, out_v,
                   in_s0, in_s1, out_s0, out_s1):
        wid = lax.axis_index("s") * NC + lax.axis_index("c")
        pltpu.sync_copy(idx_hbm, idx_v)

        def jv_pre(g, _):
            qv = g * 16 + lax.iota(jnp.int32, 16)
            for kk in range(TOPK):
                jv_v[pl.ds(kk * B + g * 16, 16)] = plsc.load_gather(
                    idx_v, [qv * TOPK + kk])
            return 0

        lax.fori_loop(0, QG, jv_pre, 0, unroll=4)
        upw = NUNITS // NW
        in_sems = (in_s0, in_s1)
        out_sems = (out_s0, out_s1)

        def unit_pe(i):
            u = wid + i * NW
            return u // (EMB // EHALF), (u % (EMB // EHALF)) * EHALF

        def in_copy(i, b):
            p, e0 = unit_pe(i)
            pltpu.async_copy(pvt_hbm.at[p, pl.ds(e0, EHALF)],
                             in_v.at[pl.ds(b * EHALF, EHALF)], in_sems[b])

        def in_wait(b):
            pltpu.make_async_copy(pvt_hbm.at[0, pl.ds(0, EHALF)],
                                  in_v.at[pl.ds(b * EHALF, EHALF)],
                                  in_sems[b]).wait()

        def out_copy(i, b):
            p, e0 = unit_pe(i)
            for kk in range(TOPK):
                pltpu.async_copy(
                    out_v.at[pl.ds((2 * b + kk) * EHALF, EHALF)],
                    out_hbm.at[kk * PLEN + p, pl.ds(e0, EHALF)], out_sems[b])

        def out_wait(b):
            for kk in range(TOPK):
                pltpu.make_async_copy(
                    out_v.at[pl.ds((2 * b + kk) * EHALF, EHALF)],
                    out_hbm.at[0, pl.ds(0, EHALF)], out_sems[b]).wait()

        def compute(b):
            LOOKAHEAD = 16

            def qg_body(g, _):
                jvs = [jv_v[pl.ds(kk * B + g * 16, 16)]
                       for kk in range(TOPK)]
                pend = []
                for t in range(TOPK * EHALF):
                    kk, e = t // EHALF, t % EHALF
                    v = plsc.load_gather(
                        in_v,
                        [jnp.full((16,), b * EHALF + e, jnp.int32), jvs[kk]])
                    pend.append((t, v))
                    if len(pend) > LOOKAHEAD:
                        r, vv = pend.pop(0)
                        out_v[2 * b * EHALF + r, pl.ds(g * 16, 16)] = vv
                for r, vv in pend:
                    out_v[2 * b * EHALF + r, pl.ds(g * 16, 16)] = vv
                return 0

            lax.fori_loop(0, QG, qg_body, 0, unroll=2)

        in_copy(0, 0)
        in_copy(1, 1)
        in_wait(0)
        compute(0)
        out_copy(0, 0)
        in_copy(2, 0)
        in_wait(1)
        compute(1)
        out_copy(1, 1)
        in_copy(3, 1)

        def pair_body(it, _):
            i0 = 2 * it
            out_wait(0)
            in_wait(0)
            compute(0)
            out_copy(i0, 0)
            in_copy(i0 + 2, 0)
            out_wait(1)
            in_wait(1)
            compute(1)
            out_copy(i0 + 1, 1)
            in_copy(jnp.minimum(i0 + 3, upw - 1), 1)
            return 0

        lax.fori_loop(1, (upw - 1) // 2, pair_body, 0)

        out_wait(0)
        in_wait(0)
        compute(0)
        out_copy(upw - 1, 0)
        in_wait(1)
        out_wait(1)
        out_wait(0)

    return _sc_gather


def kernel(inputs, prompt_keys, prompt_values):
    x = inputs.reshape(B, KD)
    kpad = jnp.pad(prompt_keys, ((0, POOL_PAD - POOL), (0, 0)))
    idx = _topk(x, kpad).reshape(TOPK * B)
    pvt = jnp.transpose(prompt_values, (1, 2, 0))
    out_t = _make_sc_gather()(idx, pvt)
    return jnp.transpose(out_t, (2, 0, 1))


# speedup vs baseline: 1.0108x; 1.0108x over previous
import functools

import jax
import jax.numpy as jnp
from jax import lax
from jax.experimental import pallas as pl
from jax.experimental.pallas import tpu as pltpu
from jax.experimental.pallas import tpu_sc as plsc

KD = 128
EMB = 64
PLEN = 200
POOL = 1000
TOPK = 2
B = 1024
POOL_PAD = 1024

NC, NS = 2, 16
NW = NC * NS
EHALF = 16
NUNITS = PLEN * (EMB // EHALF)
QG = B // 16


def _topk_body(x_ref, k_ref, idx_ref):
    x = x_ref[...]
    k = k_ref[...]
    ksq = jnp.sum(k * k, axis=1, keepdims=True)
    kn = k * lax.rsqrt(jnp.maximum(ksq, 1e-12))
    s = lax.dot_general(x, kn, (((1,), (1,)), ((), ())),
                        precision=lax.Precision.HIGHEST,
                        preferred_element_type=jnp.float32)
    cols = lax.broadcasted_iota(jnp.int32, (B, POOL_PAD), 1)
    s = jnp.where(cols >= POOL, jnp.inf, s)
    m1 = jnp.min(s, axis=1, keepdims=True)
    i1 = jnp.min(jnp.where(s == m1, cols, POOL_PAD), axis=1, keepdims=True)
    s2 = jnp.where(cols == i1, jnp.inf, s)
    m2 = jnp.min(s2, axis=1, keepdims=True)
    i2 = jnp.min(jnp.where(s2 == m2, cols, POOL_PAD), axis=1, keepdims=True)
    idx_ref[...] = jnp.concatenate([i1, i2], axis=1)


_topk = pl.pallas_call(
    _topk_body,
    out_shape=jax.ShapeDtypeStruct((B, TOPK), jnp.int32),
)


@functools.cache
def _make_sc_gather():
    mesh = plsc.VectorSubcoreMesh(core_axis_name="c", subcore_axis_name="s")

    @functools.partial(
        pl.kernel,
        mesh=mesh,
        compiler_params=pltpu.CompilerParams(needs_layout_passes=False),
        out_type=jax.ShapeDtypeStruct((TOPK * PLEN, EMB, B), jnp.float32),
        scratch_types=[
            pltpu.VMEM((TOPK * B,), jnp.int32),
            pltpu.VMEM((TOPK * B,), jnp.int32),
            pltpu.VMEM((2 * EHALF, POOL), jnp.float32),
            pltpu.VMEM((2 * TOPK * EHALF, B), jnp.float32),
            pltpu.SemaphoreType.DMA,
            pltpu.SemaphoreType.DMA,
            pltpu.SemaphoreType.DMA,
            pltpu.SemaphoreType.DMA,
        ],
    )
    def _sc_gather(idx_hbm, pvt_hbm, out_hbm, idx_v, jv_v, in_v, out_v,
                   in_s0, in_s1, out_s0, out_s1):
        wid = lax.axis_index("s") * NC + lax.axis_index("c")
        pltpu.sync_copy(idx_hbm, idx_v)

        def jv_pre(g, _):
            qv = g * 16 + lax.iota(jnp.int32, 16)
            for kk in range(TOPK):
                jv_v[pl.ds(kk * B + g * 16, 16)] = plsc.load_gather(
                    idx_v, [qv * TOPK + kk])
            return 0

        lax.fori_loop(0, QG, jv_pre, 0, unroll=4)
        upw = NUNITS // NW
        in_sems = (in_s0, in_s1)
        out_sems = (out_s0, out_s1)

        def unit_pe(i):
            u = wid + i * NW
            return u // (EMB // EHALF), (u % (EMB // EHALF)) * EHALF

        def in_copy(i, b):
            p, e0 = unit_pe(i)
            pltpu.async_copy(pvt_hbm.at[p, pl.ds(e0, EHALF)],
                             in_v.at[pl.ds(b * EHALF, EHALF)], in_sems[b])

        def in_wait(b):
            pltpu.make_async_copy(pvt_hbm.at[0, pl.ds(0, EHALF)],
                                  in_v.at[pl.ds(b * EHALF, EHALF)],
                                  in_sems[b]).wait()

        def out_copy(i, b):
            p, e0 = unit_pe(i)
            for kk in range(TOPK):
                pltpu.async_copy(
                    out_v.at[pl.ds((2 * b + kk) * EHALF, EHALF)],
                    out_hbm.at[kk * PLEN + p, pl.ds(e0, EHALF)], out_sems[b])

        def out_wait(b):
            for kk in range(TOPK):
                pltpu.make_async_copy(
                    out_v.at[pl.ds((2 * b + kk) * EHALF, EHALF)],
                    out_hbm.at[0, pl.ds(0, EHALF)], out_sems[b]).wait()

        def compute(b):
            LOOKAHEAD = 4

            def qg_body(g, _):
                jvs = [jv_v[pl.ds(kk * B + g * 16, 16)]
                       for kk in range(TOPK)]
                pend = []
                for t in range(TOPK * EHALF):
                    kk, e = t // EHALF, t % EHALF
                    v = plsc.load_gather(
                        in_v,
                        [jnp.full((16,), b * EHALF + e, jnp.int32), jvs[kk]])
                    pend.append((t, v))
                    if len(pend) > LOOKAHEAD:
                        r, vv = pend.pop(0)
                        out_v[2 * b * EHALF + r, pl.ds(g * 16, 16)] = vv
                for r, vv in pend:
                    out_v[2 * b * EHALF + r, pl.ds(g * 16, 16)] = vv
                return 0

            lax.fori_loop(0, QG, qg_body, 0, unroll=2)

        in_copy(0, 0)
        in_copy(1, 1)
        in_wait(0)
        compute(0)
        out_copy(0, 0)
        in_copy(2, 0)
        in_wait(1)
        compute(1)
        out_copy(1, 1)
        in_copy(3, 1)

        def pair_body(it, _):
            i0 = 2 * it
            out_wait(0)
            in_wait(0)
            compute(0)
            out_copy(i0, 0)
            in_copy(i0 + 2, 0)
            out_wait(1)
            in_wait(1)
            compute(1)
            out_copy(i0 + 1, 1)
            in_copy(jnp.minimum(i0 + 3, upw - 1), 1)
            return 0

        lax.fori_loop(1, (upw - 1) // 2, pair_body, 0)

        out_wait(0)
        in_wait(0)
        compute(0)
        out_copy(upw - 1, 0)
        in_wait(1)
        out_wait(1)
        out_wait(0)

    return _sc_gather


def kernel(inputs, prompt_keys, prompt_values):
    x = inputs.reshape(B, KD)
    kpad = jnp.pad(prompt_keys, ((0, POOL_PAD - POOL), (0, 0)))
    idx = _topk(x, kpad).reshape(TOPK * B)
    pvt = jnp.transpose(prompt_values, (1, 2, 0))
    out_t = _make_sc_gather()(idx, pvt)
    return jnp.transpose(out_t, (2, 0, 1))


# speedup vs baseline: 1.0359x; 1.0249x over previous
import functools

import jax
import jax.numpy as jnp
from jax import lax
from jax.experimental import pallas as pl
from jax.experimental.pallas import tpu as pltpu
from jax.experimental.pallas import tpu_sc as plsc

KD = 128
EMB = 64
PLEN = 200
POOL = 1000
TOPK = 2
B = 1024
POOL_PAD = 1024

NC, NS = 2, 16
NW = NC * NS
EHALF = 16
NUNITS = PLEN * (EMB // EHALF)
QG = B // 16


def _topk_body(x_ref, k_ref, idx_ref):
    x = x_ref[...]
    k = k_ref[...]
    ksq = jnp.sum(k * k, axis=1, keepdims=True)
    kn = k * lax.rsqrt(jnp.maximum(ksq, 1e-12))
    s = lax.dot_general(x, kn, (((1,), (1,)), ((), ())),
                        precision=lax.Precision.HIGHEST,
                        preferred_element_type=jnp.float32)
    cols = lax.broadcasted_iota(jnp.int32, (B, POOL_PAD), 1)
    s = jnp.where(cols >= POOL, jnp.inf, s)
    m1 = jnp.min(s, axis=1, keepdims=True)
    i1 = jnp.min(jnp.where(s == m1, cols, POOL_PAD), axis=1, keepdims=True)
    s2 = jnp.where(cols == i1, jnp.inf, s)
    m2 = jnp.min(s2, axis=1, keepdims=True)
    i2 = jnp.min(jnp.where(s2 == m2, cols, POOL_PAD), axis=1, keepdims=True)
    idx_ref[...] = jnp.concatenate([i1, i2], axis=1)


_topk = pl.pallas_call(
    _topk_body,
    out_shape=jax.ShapeDtypeStruct((B, TOPK), jnp.int32),
)


@functools.cache
def _make_sc_gather():
    mesh = plsc.VectorSubcoreMesh(core_axis_name="c", subcore_axis_name="s")

    @functools.partial(
        pl.kernel,
        mesh=mesh,
        compiler_params=pltpu.CompilerParams(needs_layout_passes=False),
        out_type=jax.ShapeDtypeStruct((TOPK * PLEN, EMB, B), jnp.float32),
        scratch_types=[
            pltpu.VMEM((TOPK * B,), jnp.int32),
            pltpu.VMEM((TOPK * B,), jnp.int32),
            pltpu.VMEM((2 * EHALF, POOL), jnp.float32),
            pltpu.VMEM((2 * TOPK * EHALF, B), jnp.float32),
            pltpu.SemaphoreType.DMA,
            pltpu.SemaphoreType.DMA,
            pltpu.SemaphoreType.DMA,
            pltpu.SemaphoreType.DMA,
        ],
    )
    def _sc_gather(idx_hbm, pvt_hbm, out_hbm, idx_v, jv_v, in_v, out_v,
                   in_s0, in_s1, out_s0, out_s1):
        wid = lax.axis_index("s") * NC + lax.axis_index("c")
        pltpu.sync_copy(idx_hbm, idx_v)

        def jv_pre(g, _):
            qv = g * 16 + lax.iota(jnp.int32, 16)
            for kk in range(TOPK):
                jv_v[pl.ds(kk * B + g * 16, 16)] = plsc.load_gather(
                    idx_v, [qv * TOPK + kk])
            return 0

        lax.fori_loop(0, QG, jv_pre, 0, unroll=4)
        upw = NUNITS // NW
        in_sems = (in_s0, in_s1)
        out_sems = (out_s0, out_s1)

        def unit_pe(i):
            u = wid + i * NW
            return u // (EMB // EHALF), (u % (EMB // EHALF)) * EHALF

        def in_copy(i, b):
            p, e0 = unit_pe(i)
            pltpu.async_copy(pvt_hbm.at[p, pl.ds(e0, EHALF)],
                             in_v.at[pl.ds(b * EHALF, EHALF)], in_sems[b])

        def in_wait(b):
            pltpu.make_async_copy(pvt_hbm.at[0, pl.ds(0, EHALF)],
                                  in_v.at[pl.ds(b * EHALF, EHALF)],
                                  in_sems[b]).wait()

        def out_copy(i, b):
            p, e0 = unit_pe(i)
            for kk in range(TOPK):
                pltpu.async_copy(
                    out_v.at[pl.ds((2 * b + kk) * EHALF, EHALF)],
                    out_hbm.at[kk * PLEN + p, pl.ds(e0, EHALF)], out_sems[b])

        def out_wait(b):
            for kk in range(TOPK):
                pltpu.make_async_copy(
                    out_v.at[pl.ds((2 * b + kk) * EHALF, EHALF)],
                    out_hbm.at[0, pl.ds(0, EHALF)], out_sems[b]).wait()

        def compute(b):
            LOOKAHEAD = 8

            def qg_body(g, _):
                jvs = [jv_v[pl.ds(kk * B + g * 16, 16)]
                       for kk in range(TOPK)]
                pend = []
                for t in range(TOPK * EHALF):
                    kk, e = t // EHALF, t % EHALF
                    v = plsc.load_gather(
                        in_v,
                        [jnp.full((16,), b * EHALF + e, jnp.int32), jvs[kk]])
                    pend.append((t, v))
                    if len(pend) > LOOKAHEAD:
                        r, vv = pend.pop(0)
                        out_v[2 * b * EHALF + r, pl.ds(g * 16, 16)] = vv
                for r, vv in pend:
                    out_v[2 * b * EHALF + r, pl.ds(g * 16, 16)] = vv
                return 0

            lax.fori_loop(0, QG, qg_body, 0, unroll=2)

        in_copy(0, 0)
        in_copy(1, 1)
        in_wait(0)
        compute(0)
        out_copy(0, 0)
        in_copy(2, 0)
        in_wait(1)
        compute(1)
        out_copy(1, 1)
        in_copy(3, 1)

        def pair_body(it, _):
            i0 = 2 * it
            out_wait(0)
            in_wait(0)
            compute(0)
            out_copy(i0, 0)
            in_copy(i0 + 2, 0)
            out_wait(1)
            in_wait(1)
            compute(1)
            out_copy(i0 + 1, 1)
            in_copy(jnp.minimum(i0 + 3, upw - 1), 1)
            return 0

        lax.fori_loop(1, (upw - 1) // 2, pair_body, 0)

        out_wait(0)
        in_wait(0)
        compute(0)
        out_copy(upw - 1, 0)
        in_wait(1)
        out_wait(1)
        out_wait(0)

    return _sc_gather


def kernel(inputs, prompt_keys, prompt_values):
    x = inputs.reshape(B, KD)
    kpad = jnp.pad(prompt_keys, ((0, POOL_PAD - POOL), (0, 0)))
    idx = _topk(x, kpad).reshape(TOPK * B)
    pvt = jnp.transpose(prompt_values, (1, 2, 0))
    out_t = _make_sc_gather()(idx, pvt)
    return jnp.transpose(out_t, (2, 0, 1))
